# baseline (speedup 1.0000x reference)
import functools

import jax
import jax.numpy as jnp
from jax import lax
from jax.experimental import pallas as pl
from jax.experimental.pallas import tpu as pltpu
from jax.experimental.pallas import tpu_sc as plsc

V = 200
D = 98304
NDEST = 1600
NPAD = NDEST + 16
SUB = 4096
NSUB = D // SUB
FA = 832
NB = NDEST - FA
NSEG = NSUB // 2
APT = FA // 16
ABLK = 3
ATAIL = APT - 16 * ABLK
BGRP = 8
BPW = NB // 16
GSEG = BPW // BGRP


def _make_sc_call():
    mesh = plsc.VectorSubcoreMesh(core_axis_name="c", subcore_axis_name="s")

    @functools.partial(
        pl.kernel,
        mesh=mesh,
        out_type=jax.ShapeDtypeStruct((NDEST, NSUB, 32, 128), jnp.float32),
        scratch_types=[
            pltpu.VMEM((NPAD,), jnp.int32),
            pltpu.VMEM((BPW + 16,), jnp.int32),
            pltpu.VMEM((NSEG * BGRP,), jnp.int32),
            pltpu.VMEM((2, BGRP, 32, 128), jnp.float32),
            pltpu.VMEM_SHARED((V, 32, 128), jnp.float32),
            pltpu.SemaphoreType.DMA,
            pltpu.SemaphoreType.DMA,
            pltpu.SemaphoreType.DMA,
            pltpu.SemaphoreType.DMA,
            pltpu.SemaphoreType.DMA,
            pltpu.SemaphoreType.DMA,
        ],
    )
    def sc_gather(idx_hbm, idxb_hbm, srcb0_hbm, table_hbm, out_hbm,
                  idx_v, idxb_v, idxb0_v, bbufs, stage,
                  asem, stsem, bg0, bg1, bs0, bs1):
        bgsems = (bg0, bg1)
        bssems = (bs0, bs1)
        c = lax.axis_index("c")
        s = lax.axis_index("s")
        pltpu.sync_copy(idx_hbm, idx_v)
        pltpu.sync_copy(idxb_hbm.at[pl.ds(s * BPW, BPW)], idxb_v.at[pl.ds(0, BPW)])
        pltpu.sync_copy(
            srcb0_hbm.at[pl.ds((s * 2 + c) * NSEG * BGRP, NSEG * BGRP)], idxb0_v
        )

        def a_start(d, gc, row):
            pltpu.make_async_copy(
                stage.at[row], out_hbm.at[d, gc], asem
            ).start()

        def a_drain(n):
            pltpu.make_async_copy(
                table_hbm.at[pl.ds(0, n)], out_hbm.at[pl.ds(0, n), 0], asem
            ).wait()

        def b_dst(t, gc):
            return out_hbm.at[pl.ds(FA + s * BPW + t * BGRP, BGRP), gc]

        def b_gather_start(t, b):
            vb = idxb_v[pl.ds(t * BGRP, 16)]
            for j in range(BGRP):
                pltpu.make_async_copy(
                    stage.at[vb[j]], bbufs.at[b, j], bgsems[b]
                ).start()

        def b_gather_wait(b):
            pltpu.make_async_copy(
                stage.at[pl.ds(0, BGRP)], bbufs.at[b], bgsems[b]
            ).wait()

        def b_scatter_start(t, gc, b):
            pltpu.make_async_copy(bbufs.at[b], b_dst(t, gc), bssems[b]).start()

        def b_scatter_wait(t, gc, b):
            pltpu.make_async_copy(bbufs.at[b], b_dst(t, gc), bssems[b]).wait()

        for gseg in range(NSEG):
            gc = c * NSEG + gseg

            if gseg > 0:
                b_scatter_wait(GSEG - 2, gc - 1, (GSEG - 2) % 2)

            pltpu.make_async_copy(
                table_hbm.at[idxb0_v.at[pl.ds(gseg * BGRP, BGRP)]],
                bbufs.at[0],
                bgsems[0],
            ).start()

            cnt = jnp.where(s < 8, 13, 12)
            rstart = s * 12 + jnp.minimum(s, 8)

            def st_issue(i, carry):
                r = rstart + i
                pltpu.make_async_copy(
                    table_hbm.at[r * NSUB + gc], stage.at[r], stsem
                ).start()
                return carry

            def st_drain(i, carry):
                pltpu.make_async_copy(
                    table_hbm.at[0], stage.at[0], stsem
                ).wait()
                return carry

            lax.fori_loop(0, cnt, st_issue, 0)
            lax.fori_loop(0, cnt, st_drain, 0)
            plsc.subcore_barrier()

            for t in range(GSEG):
                if t + 1 < GSEG:
                    if t >= 1:
                        b_scatter_wait(t - 1, gc, (t - 1) % 2)
                    elif gseg > 0:
                        b_scatter_wait(GSEG - 1, gc - 1, (GSEG - 1) % 2)
                    b_gather_start(t + 1, (t + 1) % 2)

                if t < ABLK:
                    d0 = s * APT + t * 16
                    v16 = idx_v[pl.ds(d0, 16)]
                    for j in range(16):
                        a_start(d0 + j, gc, v16[j])
                elif t == ABLK:
                    d0 = s * APT + ABLK * 16
                    vt = idx_v[pl.ds(d0, 16)]
                    for j in range(ATAIL):
                        a_start(d0 + j, gc, vt[j])
                else:
                    a_drain(16)

                b_gather_wait(t % 2)
                b_scatter_start(t, gc, t % 2)

            a_drain(APT - 16 * (GSEG - ABLK - 1))
            plsc.subcore_barrier()

        gclast = c * NSEG + NSEG - 1
        b_scatter_wait(GSEG - 2, gclast, (GSEG - 2) % 2)
        b_scatter_wait(GSEG - 1, gclast, (GSEG - 1) % 2)

    return sc_gather


_SC_GATHER = _make_sc_call()


def kernel(prefix, embedding):
    B, S = prefix.shape
    idx = prefix.reshape(B * S).astype(jnp.int32)
    idx_pad = jnp.concatenate([idx, jnp.zeros((NPAD - NDEST,), jnp.int32)])
    w = jnp.arange(32, dtype=jnp.int32)
    s_ = (w // 2)[:, None, None]
    c_ = (w % 2)[:, None, None]
    gs = jnp.arange(NSEG, dtype=jnp.int32)[None, :, None]
    jj = jnp.arange(BGRP, dtype=jnp.int32)[None, None, :]
    srcb0 = (idx[FA + s_ * BPW + jj] * NSUB + c_ * NSEG + gs).reshape(-1)
    table = embedding.reshape(V * NSUB, 32, 128)
    out = _SC_GATHER(idx_pad, idx[FA:], srcb0, table)
    return out.reshape(B, S, D)
